# baseline (speedup 1.0000x reference)
import functools

import jax
import jax.numpy as jnp
from jax.experimental import pallas as pl
from jax.experimental.pallas import tpu as pltpu


def _bq_matmul_kernel(x_ref, binary_ref, scale_ref, bias_ref, out_ref):
    nbits, g, nf = binary_ref.shape
    combo = binary_ref[0]
    for b in range(1, nbits):
        combo = combo | (binary_ref[b] << (8 * b))
    ncombo = (~combo)[:, None, :]
    qq = 2 * jax.lax.broadcasted_iota(jnp.int32, (1, 4, 1), 1)
    signbit = jnp.int32(-2**31)
    w = None
    for b in range(nbits):
        sb = jnp.broadcast_to(scale_ref[b].astype(jnp.bfloat16), (2, nf))
        spair = pltpu.bitcast(sb, jnp.int32)
        flip = (((ncombo << (8 - 8 * b + qq)) & jnp.int32(0x8000))
                | ((ncombo << (25 - 8 * b + qq)) & signbit))
        wb = pltpu.bitcast(flip ^ spair[None], jnp.bfloat16)
        w = wb if w is None else w + wb
    weff = w.reshape(g * 8, nf)
    xb = x_ref[...].astype(jnp.bfloat16)
    out = jnp.dot(xb, weff, preferred_element_type=jnp.float32)
    out_ref[...] = out + bias_ref[...]


@functools.partial(jax.jit, static_argnames=())
def kernel(x, binary, scale, bias):
    size_out = x.shape[:-1] + (bias.shape[-1],)
    x2 = x.reshape(-1, x.shape[-1])
    t, nx = x2.shape
    nbits = scale.shape[1]
    nf = scale.shape[2]
    g = nx // 8
    binary3 = binary.reshape(nbits, g, nf)
    scale2 = scale.reshape(nbits, nf)
    bias2 = bias.reshape(1, nf)
    out = pl.pallas_call(
        _bq_matmul_kernel,
        out_shape=jax.ShapeDtypeStruct((t, nf), jnp.float32),
    )(x2, binary3, scale2, bias2)
    return out.reshape(size_out)


# speedup vs baseline: 1.2069x; 1.2069x over previous
import functools

import jax
import jax.numpy as jnp
from jax.experimental import pallas as pl


def _bq_matmul_kernel(x_ref, binary_ref, scale_ref, bias_ref, out_ref):
    nbits, g, nf = binary_ref.shape
    combo = binary_ref[0]
    for b in range(1, nbits):
        combo = combo | (binary_ref[b] << (8 * b))
    ncombo = (~combo)[:, None, :]
    ii = jax.lax.broadcasted_iota(jnp.int32, (1, 8, 1), 1)
    signbit = jnp.int32(-2**31)
    t = ncombo << (16 + ii)
    w = None
    for b in range(nbits):
        flip = (t << (8 * (1 - b))) & signbit
        sint = jax.lax.bitcast_convert_type(scale_ref[b], jnp.int32)
        wb = jax.lax.bitcast_convert_type(flip ^ sint[None, None, :],
                                          jnp.float32)
        w = wb if w is None else w + wb
    weff = w.reshape(g * 8, nf).astype(jnp.bfloat16)
    xb = x_ref[...].astype(jnp.bfloat16)
    out = jnp.dot(xb, weff, preferred_element_type=jnp.float32)
    out_ref[...] = out + bias_ref[...]


@functools.partial(jax.jit, static_argnames=())
def kernel(x, binary, scale, bias):
    size_out = x.shape[:-1] + (bias.shape[-1],)
    x2 = x.reshape(-1, x.shape[-1])
    t, nx = x2.shape
    nbits = scale.shape[1]
    nf = scale.shape[2]
    g = nx // 8
    binary3 = binary.reshape(nbits, g, nf)
    scale2 = scale.reshape(nbits, nf)
    bias2 = bias.reshape(1, nf)
    out = pl.pallas_call(
        _bq_matmul_kernel,
        out_shape=jax.ShapeDtypeStruct((t, nf), jnp.float32),
    )(x2, binary3, scale2, bias2)
    return out.reshape(size_out)
